# baseline (speedup 1.0000x reference)
import functools
import numpy as np

import jax
import jax.numpy as jnp
from jax import lax
from jax.experimental import pallas as pl
from jax.experimental.pallas import tpu as pltpu
from jax.experimental.pallas import tpu_sc as plsc

D_MODEL = 4096
N_EXPERTS = 64
TOP_K = 8
N_CAND = 32
EPS = 1e-6
TB = 256
N_TOKENS = 8192
NW = 32
NCHUNKS = 1
CT = N_TOKENS // NCHUNKS
NEG = -3.4e38


def _tc_body(h_ref, weff_ref, wcat_ref, b_ref, probs_ref, bvh_ref):
    x = h_ref[...]
    var = jnp.mean(x * x, axis=-1, keepdims=True)
    xn = x * lax.rsqrt(var + EPS) * weff_ref[...]
    logits = jnp.dot(xn, wcat_ref[...], preferred_element_type=jnp.float32)
    bvh_ref[...] = logits[:, :N_EXPERTS]
    rl = logits[:, N_EXPERTS:] + b_ref[...]
    m = jnp.max(rl, axis=-1, keepdims=True)
    e = jnp.exp(rl - m)
    probs_ref[...] = e / jnp.sum(e, axis=-1, keepdims=True)


def _rev(x):
    return lax.rev(x, dimensions=(0,))


def _vsort(x):
    return lax.sort(x, dimension=0, is_stable=False, num_keys=1)


def _make_sc_body(tpw):
    def _sc_body(bvh_hbm, probs_hbm, pes_hbm, w_hbm, i_hbm,
                 bvh_v, probs_v, pes_v, w_v, i_v):
        wid = lax.axis_index("s") * 2 + lax.axis_index("c")
        base = wid * tpw * N_EXPERTS
        obase = wid * tpw * 16
        pltpu.sync_copy(bvh_hbm.at[pl.ds(base, tpw * N_EXPERTS)], bvh_v)
        pltpu.sync_copy(probs_hbm.at[pl.ds(base, tpw * N_EXPERTS)], probs_v)
        pltpu.sync_copy(pes_hbm, pes_v)

        iota = lax.broadcasted_iota(jnp.int32, (16,), 0)
        ids = [iota + (16 * k) for k in range(4)]

        @plsc.parallel_loop(0, tpw, 1, unroll=2)
        def token_body(t):
            tb = t * N_EXPERTS
            b = [bvh_v[pl.ds(tb + 16 * k, 16)] for k in range(4)]
            g = [probs_v[pl.ds(tb + 16 * k, 16)] for k in range(4)]

            s = [_vsort(b[k]) for k in range(4)]
            lo01 = _vsort(jnp.minimum(s[0], _rev(s[1])))
            hi01 = _vsort(jnp.maximum(s[0], _rev(s[1])))
            lo23 = _vsort(jnp.minimum(s[2], _rev(s[3])))
            hi23 = _vsort(jnp.maximum(s[2], _rev(s[3])))
            u0 = jnp.maximum(lo01, _rev(hi23))
            u1 = jnp.maximum(hi01, _rev(lo23))
            thr = jnp.min(jnp.minimum(u0, u1))

            p = [jnp.where(b[k] >= thr, g[k], -1.0) for k in range(4)]
            out_v = jnp.zeros((16,), jnp.float32)
            out_i = jnp.zeros((16,), jnp.int32)
            for r in range(TOP_K):
                mx = jnp.max(jnp.maximum(jnp.maximum(p[0], p[1]),
                                         jnp.maximum(p[2], p[3])))
                sel = [p[k] == mx for k in range(4)]
                tb_ = [jnp.where(sel[k], b[k], NEG) for k in range(4)]
                tm = jnp.max(jnp.maximum(jnp.maximum(tb_[0], tb_[1]),
                                         jnp.maximum(tb_[2], tb_[3])))
                sel2 = [jnp.logical_and(sel[k], b[k] == tm) for k in range(4)]
                idx = jnp.max(jnp.maximum(
                    jnp.maximum(jnp.where(sel2[0], ids[0], -1),
                                jnp.where(sel2[1], ids[1], -1)),
                    jnp.maximum(jnp.where(sel2[2], ids[2], -1),
                                jnp.where(sel2[3], ids[3], -1))))
                slot = iota == r
                out_v = jnp.where(slot, mx, out_v)
                out_i = jnp.where(slot, idx, out_i)
                p = [jnp.where(sel2[k], -1.0, p[k]) for k in range(4)]

            valid = iota < TOP_K
            total = jnp.sum(jnp.where(valid, out_v, 0.0))
            pes_g = plsc.load_gather(pes_v, [out_i])
            w = out_v / total * pes_g
            w_v[pl.ds(t * 16, 16)] = w
            i_v[pl.ds(t * 16, 16)] = out_i

        pltpu.sync_copy(w_v, w_hbm.at[pl.ds(obase, tpw * 16)])
        pltpu.sync_copy(i_v, i_hbm.at[pl.ds(obase, tpw * 16)])

    return _sc_body


def _make_sc_route(nt):
    tpw = nt // NW
    return pl.kernel(
        _make_sc_body(tpw),
        mesh=plsc.VectorSubcoreMesh(core_axis_name="c", subcore_axis_name="s"),
        out_type=[
            jax.ShapeDtypeStruct((nt * 16,), jnp.float32),
            jax.ShapeDtypeStruct((nt * 16,), jnp.int32),
        ],
        scratch_types=[
            pltpu.VMEM((tpw * N_EXPERTS,), jnp.float32),
            pltpu.VMEM((tpw * N_EXPERTS,), jnp.float32),
            pltpu.VMEM((N_EXPERTS,), jnp.float32),
            pltpu.VMEM((tpw * 16,), jnp.float32),
            pltpu.VMEM((tpw * 16,), jnp.int32),
        ],
        compiler_params=pltpu.CompilerParams(needs_layout_passes=False),
    )


_sc_route = _make_sc_route(CT)


def _tc_dense(h, weff, wcat, b2):
    nt = h.shape[0]
    grid = (nt // TB,)
    return pl.pallas_call(
        _tc_body,
        grid=grid,
        in_specs=[
            pl.BlockSpec((TB, D_MODEL), lambda i: (i, 0)),
            pl.BlockSpec((1, D_MODEL), lambda i: (0, 0)),
            pl.BlockSpec((D_MODEL, 2 * N_EXPERTS), lambda i: (0, 0)),
            pl.BlockSpec((1, N_EXPERTS), lambda i: (0, 0)),
        ],
        out_specs=[
            pl.BlockSpec((TB, N_EXPERTS), lambda i: (i, 0)),
            pl.BlockSpec((TB, N_EXPERTS), lambda i: (i, 0)),
        ],
        out_shape=[
            jax.ShapeDtypeStruct((nt, N_EXPERTS), jnp.float32),
            jax.ShapeDtypeStruct((nt, N_EXPERTS), jnp.float32),
        ],
    )(h, weff, wcat, b2)


@jax.jit
def kernel(hidden_states, norm_weight, scale, W_proj, b_proj, W_bvh,
           per_expert_scale):
    weff = (norm_weight * scale * np.float32(np.sqrt(D_MODEL))).reshape(1, D_MODEL)
    wcat = jnp.concatenate([W_bvh, W_proj], axis=1)
    b2 = b_proj.reshape(1, N_EXPERTS)

    probs_c, w_c, i_c = [], [], []
    dense = []
    for c in range(NCHUNKS):
        h = lax.slice_in_dim(hidden_states, c * CT, (c + 1) * CT, axis=0)
        dense.append(_tc_dense(h, weff, wcat, b2))
    for c in range(NCHUNKS):
        probs, bvh = dense[c]
        w16, i16 = _sc_route(bvh.reshape(-1), probs.reshape(-1),
                             per_expert_scale)
        probs_c.append(probs)
        w_c.append(w16.reshape(CT, 16)[:, :TOP_K])
        i_c.append(i16.reshape(CT, 16)[:, :TOP_K])

    return (jnp.concatenate(probs_c, axis=0),
            jnp.concatenate(w_c, axis=0),
            jnp.concatenate(i_c, axis=0))


# speedup vs baseline: 1.0341x; 1.0341x over previous
import functools
import numpy as np

import jax
import jax.numpy as jnp
from jax import lax
from jax.experimental import pallas as pl
from jax.experimental.pallas import tpu as pltpu
from jax.experimental.pallas import tpu_sc as plsc

D_MODEL = 4096
N_EXPERTS = 64
TOP_K = 8
N_CAND = 32
EPS = 1e-6
TB = 512
N_TOKENS = 8192
NW = 32
NCHUNKS = 1
CT = N_TOKENS // NCHUNKS
NEG = -3.4e38


def _tc_body(h_ref, weff_ref, wcat_ref, b_ref, probs_ref, bvh_ref):
    x = h_ref[...]
    var = jnp.mean(x * x, axis=-1, keepdims=True)
    xn = x * lax.rsqrt(var + EPS) * weff_ref[...]
    logits = jnp.dot(xn, wcat_ref[...], preferred_element_type=jnp.float32)
    bvh_ref[...] = logits[:, :N_EXPERTS]
    rl = logits[:, N_EXPERTS:] + b_ref[...]
    m = jnp.max(rl, axis=-1, keepdims=True)
    e = jnp.exp(rl - m)
    probs_ref[...] = e / jnp.sum(e, axis=-1, keepdims=True)


def _rev(x):
    return lax.rev(x, dimensions=(0,))


def _vsort(x):
    return lax.sort(x, dimension=0, is_stable=False, num_keys=1)


def _make_sc_body(tpw):
    def _sc_body(bvh_hbm, probs_hbm, pes_hbm, w_hbm, i_hbm,
                 bvh_v, probs_v, pes_v, w_v, i_v):
        wid = lax.axis_index("s") * 2 + lax.axis_index("c")
        base = wid * tpw * N_EXPERTS
        obase = wid * tpw * 16
        pltpu.sync_copy(bvh_hbm.at[pl.ds(base, tpw * N_EXPERTS)], bvh_v)
        pltpu.sync_copy(probs_hbm.at[pl.ds(base, tpw * N_EXPERTS)], probs_v)
        pltpu.sync_copy(pes_hbm, pes_v)

        iota = lax.broadcasted_iota(jnp.int32, (16,), 0)
        ids = [iota + (16 * k) for k in range(4)]

        @plsc.parallel_loop(0, tpw, 1, unroll=2)
        def token_body(t):
            tb = t * N_EXPERTS
            b = [bvh_v[pl.ds(tb + 16 * k, 16)] for k in range(4)]
            g = [probs_v[pl.ds(tb + 16 * k, 16)] for k in range(4)]

            s = [_vsort(b[k]) for k in range(4)]
            lo01 = _vsort(jnp.minimum(s[0], _rev(s[1])))
            hi01 = _vsort(jnp.maximum(s[0], _rev(s[1])))
            lo23 = _vsort(jnp.minimum(s[2], _rev(s[3])))
            hi23 = _vsort(jnp.maximum(s[2], _rev(s[3])))
            u0 = jnp.maximum(lo01, _rev(hi23))
            u1 = jnp.maximum(hi01, _rev(lo23))
            thr = jnp.min(jnp.minimum(u0, u1))

            p = [jnp.where(b[k] >= thr, g[k], -1.0) for k in range(4)]
            out_v = jnp.zeros((16,), jnp.float32)
            out_i = jnp.zeros((16,), jnp.int32)
            for r in range(TOP_K):
                mx = jnp.max(jnp.maximum(jnp.maximum(p[0], p[1]),
                                         jnp.maximum(p[2], p[3])))
                sel = [p[k] == mx for k in range(4)]
                tb_ = [jnp.where(sel[k], b[k], NEG) for k in range(4)]
                tm = jnp.max(jnp.maximum(jnp.maximum(tb_[0], tb_[1]),
                                         jnp.maximum(tb_[2], tb_[3])))
                sel2 = [jnp.logical_and(sel[k], b[k] == tm) for k in range(4)]
                idx = jnp.max(jnp.maximum(
                    jnp.maximum(jnp.where(sel2[0], ids[0], -1),
                                jnp.where(sel2[1], ids[1], -1)),
                    jnp.maximum(jnp.where(sel2[2], ids[2], -1),
                                jnp.where(sel2[3], ids[3], -1))))
                slot = iota == r
                out_v = jnp.where(slot, mx, out_v)
                out_i = jnp.where(slot, idx, out_i)
                p = [jnp.where(sel2[k], -1.0, p[k]) for k in range(4)]

            valid = iota < TOP_K
            total = jnp.sum(jnp.where(valid, out_v, 0.0))
            pes_g = plsc.load_gather(pes_v, [out_i])
            w = out_v / total * pes_g
            w_v[pl.ds(t * 16, 16)] = w
            i_v[pl.ds(t * 16, 16)] = out_i

        pltpu.sync_copy(w_v, w_hbm.at[pl.ds(obase, tpw * 16)])
        pltpu.sync_copy(i_v, i_hbm.at[pl.ds(obase, tpw * 16)])

    return _sc_body


def _make_sc_route(nt):
    tpw = nt // NW
    return pl.kernel(
        _make_sc_body(tpw),
        mesh=plsc.VectorSubcoreMesh(core_axis_name="c", subcore_axis_name="s"),
        out_type=[
            jax.ShapeDtypeStruct((nt * 16,), jnp.float32),
            jax.ShapeDtypeStruct((nt * 16,), jnp.int32),
        ],
        scratch_types=[
            pltpu.VMEM((tpw * N_EXPERTS,), jnp.float32),
            pltpu.VMEM((tpw * N_EXPERTS,), jnp.float32),
            pltpu.VMEM((N_EXPERTS,), jnp.float32),
            pltpu.VMEM((tpw * 16,), jnp.float32),
            pltpu.VMEM((tpw * 16,), jnp.int32),
        ],
        compiler_params=pltpu.CompilerParams(needs_layout_passes=False),
    )


_sc_route = _make_sc_route(CT)


def _tc_dense(h, weff, wcat, b2):
    nt = h.shape[0]
    grid = (nt // TB,)
    return pl.pallas_call(
        _tc_body,
        grid=grid,
        in_specs=[
            pl.BlockSpec((TB, D_MODEL), lambda i: (i, 0)),
            pl.BlockSpec((1, D_MODEL), lambda i: (0, 0)),
            pl.BlockSpec((D_MODEL, 2 * N_EXPERTS), lambda i: (0, 0)),
            pl.BlockSpec((1, N_EXPERTS), lambda i: (0, 0)),
        ],
        out_specs=[
            pl.BlockSpec((TB, N_EXPERTS), lambda i: (i, 0)),
            pl.BlockSpec((TB, N_EXPERTS), lambda i: (i, 0)),
        ],
        out_shape=[
            jax.ShapeDtypeStruct((nt, N_EXPERTS), jnp.float32),
            jax.ShapeDtypeStruct((nt, N_EXPERTS), jnp.float32),
        ],
    )(h, weff, wcat, b2)


@jax.jit
def kernel(hidden_states, norm_weight, scale, W_proj, b_proj, W_bvh,
           per_expert_scale):
    weff = (norm_weight * scale * np.float32(np.sqrt(D_MODEL))).reshape(1, D_MODEL)
    wcat = jnp.concatenate([W_bvh, W_proj], axis=1)
    b2 = b_proj.reshape(1, N_EXPERTS)

    probs_c, w_c, i_c = [], [], []
    dense = []
    for c in range(NCHUNKS):
        h = lax.slice_in_dim(hidden_states, c * CT, (c + 1) * CT, axis=0)
        dense.append(_tc_dense(h, weff, wcat, b2))
    for c in range(NCHUNKS):
        probs, bvh = dense[c]
        w16, i16 = _sc_route(bvh.reshape(-1), probs.reshape(-1),
                             per_expert_scale)
        probs_c.append(probs)
        w_c.append(w16.reshape(CT, 16)[:, :TOP_K])
        i_c.append(i16.reshape(CT, 16)[:, :TOP_K])

    return (jnp.concatenate(probs_c, axis=0),
            jnp.concatenate(w_c, axis=0),
            jnp.concatenate(i_c, axis=0))


# speedup vs baseline: 1.0656x; 1.0305x over previous
import functools
import numpy as np

import jax
import jax.numpy as jnp
from jax import lax
from jax.experimental import pallas as pl
from jax.experimental.pallas import tpu as pltpu
from jax.experimental.pallas import tpu_sc as plsc

D_MODEL = 4096
N_EXPERTS = 64
TOP_K = 8
N_CAND = 32
EPS = 1e-6
TB = 1024
N_TOKENS = 8192
NW = 32
NCHUNKS = 1
CT = N_TOKENS // NCHUNKS
NEG = -3.4e38


def _tc_body(h_ref, weff_ref, wcat_ref, b_ref, probs_ref, bvh_ref):
    x = h_ref[...]
    var = jnp.mean(x * x, axis=-1, keepdims=True)
    xn = x * lax.rsqrt(var + EPS) * weff_ref[...]
    logits = jnp.dot(xn, wcat_ref[...], preferred_element_type=jnp.float32)
    bvh_ref[...] = logits[:, :N_EXPERTS]
    rl = logits[:, N_EXPERTS:] + b_ref[...]
    m = jnp.max(rl, axis=-1, keepdims=True)
    e = jnp.exp(rl - m)
    probs_ref[...] = e / jnp.sum(e, axis=-1, keepdims=True)


def _rev(x):
    return lax.rev(x, dimensions=(0,))


def _vsort(x):
    return lax.sort(x, dimension=0, is_stable=False, num_keys=1)


def _make_sc_body(tpw):
    def _sc_body(bvh_hbm, probs_hbm, pes_hbm, w_hbm, i_hbm,
                 bvh_v, probs_v, pes_v, w_v, i_v):
        wid = lax.axis_index("s") * 2 + lax.axis_index("c")
        base = wid * tpw * N_EXPERTS
        obase = wid * tpw * 16
        pltpu.sync_copy(bvh_hbm.at[pl.ds(base, tpw * N_EXPERTS)], bvh_v)
        pltpu.sync_copy(probs_hbm.at[pl.ds(base, tpw * N_EXPERTS)], probs_v)
        pltpu.sync_copy(pes_hbm, pes_v)

        iota = lax.broadcasted_iota(jnp.int32, (16,), 0)
        ids = [iota + (16 * k) for k in range(4)]

        @plsc.parallel_loop(0, tpw, 1, unroll=2)
        def token_body(t):
            tb = t * N_EXPERTS
            b = [bvh_v[pl.ds(tb + 16 * k, 16)] for k in range(4)]
            g = [probs_v[pl.ds(tb + 16 * k, 16)] for k in range(4)]

            s = [_vsort(b[k]) for k in range(4)]
            lo01 = _vsort(jnp.minimum(s[0], _rev(s[1])))
            hi01 = _vsort(jnp.maximum(s[0], _rev(s[1])))
            lo23 = _vsort(jnp.minimum(s[2], _rev(s[3])))
            hi23 = _vsort(jnp.maximum(s[2], _rev(s[3])))
            u0 = jnp.maximum(lo01, _rev(hi23))
            u1 = jnp.maximum(hi01, _rev(lo23))
            thr = jnp.min(jnp.minimum(u0, u1))

            p = [jnp.where(b[k] >= thr, g[k], -1.0) for k in range(4)]
            out_v = jnp.zeros((16,), jnp.float32)
            out_i = jnp.zeros((16,), jnp.int32)
            for r in range(TOP_K):
                mx = jnp.max(jnp.maximum(jnp.maximum(p[0], p[1]),
                                         jnp.maximum(p[2], p[3])))
                sel = [p[k] == mx for k in range(4)]
                tb_ = [jnp.where(sel[k], b[k], NEG) for k in range(4)]
                tm = jnp.max(jnp.maximum(jnp.maximum(tb_[0], tb_[1]),
                                         jnp.maximum(tb_[2], tb_[3])))
                sel2 = [jnp.logical_and(sel[k], b[k] == tm) for k in range(4)]
                idx = jnp.max(jnp.maximum(
                    jnp.maximum(jnp.where(sel2[0], ids[0], -1),
                                jnp.where(sel2[1], ids[1], -1)),
                    jnp.maximum(jnp.where(sel2[2], ids[2], -1),
                                jnp.where(sel2[3], ids[3], -1))))
                slot = iota == r
                out_v = jnp.where(slot, mx, out_v)
                out_i = jnp.where(slot, idx, out_i)
                p = [jnp.where(sel2[k], -1.0, p[k]) for k in range(4)]

            valid = iota < TOP_K
            total = jnp.sum(jnp.where(valid, out_v, 0.0))
            pes_g = plsc.load_gather(pes_v, [out_i])
            w = out_v / total * pes_g
            w_v[pl.ds(t * 16, 16)] = w
            i_v[pl.ds(t * 16, 16)] = out_i

        pltpu.sync_copy(w_v, w_hbm.at[pl.ds(obase, tpw * 16)])
        pltpu.sync_copy(i_v, i_hbm.at[pl.ds(obase, tpw * 16)])

    return _sc_body


def _make_sc_route(nt):
    tpw = nt // NW
    return pl.kernel(
        _make_sc_body(tpw),
        mesh=plsc.VectorSubcoreMesh(core_axis_name="c", subcore_axis_name="s"),
        out_type=[
            jax.ShapeDtypeStruct((nt * 16,), jnp.float32),
            jax.ShapeDtypeStruct((nt * 16,), jnp.int32),
        ],
        scratch_types=[
            pltpu.VMEM((tpw * N_EXPERTS,), jnp.float32),
            pltpu.VMEM((tpw * N_EXPERTS,), jnp.float32),
            pltpu.VMEM((N_EXPERTS,), jnp.float32),
            pltpu.VMEM((tpw * 16,), jnp.float32),
            pltpu.VMEM((tpw * 16,), jnp.int32),
        ],
        compiler_params=pltpu.CompilerParams(needs_layout_passes=False),
    )


_sc_route = _make_sc_route(CT)


def _tc_dense(h, weff, wcat, b2):
    nt = h.shape[0]
    grid = (nt // TB,)
    return pl.pallas_call(
        _tc_body,
        grid=grid,
        in_specs=[
            pl.BlockSpec((TB, D_MODEL), lambda i: (i, 0)),
            pl.BlockSpec((1, D_MODEL), lambda i: (0, 0)),
            pl.BlockSpec((D_MODEL, 2 * N_EXPERTS), lambda i: (0, 0)),
            pl.BlockSpec((1, N_EXPERTS), lambda i: (0, 0)),
        ],
        out_specs=[
            pl.BlockSpec((TB, N_EXPERTS), lambda i: (i, 0)),
            pl.BlockSpec((TB, N_EXPERTS), lambda i: (i, 0)),
        ],
        out_shape=[
            jax.ShapeDtypeStruct((nt, N_EXPERTS), jnp.float32),
            jax.ShapeDtypeStruct((nt, N_EXPERTS), jnp.float32),
        ],
    )(h, weff, wcat, b2)


@jax.jit
def kernel(hidden_states, norm_weight, scale, W_proj, b_proj, W_bvh,
           per_expert_scale):
    weff = (norm_weight * scale * np.float32(np.sqrt(D_MODEL))).reshape(1, D_MODEL)
    wcat = jnp.concatenate([W_bvh, W_proj], axis=1)
    b2 = b_proj.reshape(1, N_EXPERTS)

    probs_c, w_c, i_c = [], [], []
    dense = []
    for c in range(NCHUNKS):
        h = lax.slice_in_dim(hidden_states, c * CT, (c + 1) * CT, axis=0)
        dense.append(_tc_dense(h, weff, wcat, b2))
    for c in range(NCHUNKS):
        probs, bvh = dense[c]
        w16, i16 = _sc_route(bvh.reshape(-1), probs.reshape(-1),
                             per_expert_scale)
        probs_c.append(probs)
        w_c.append(w16.reshape(CT, 16)[:, :TOP_K])
        i_c.append(i16.reshape(CT, 16)[:, :TOP_K])

    return (jnp.concatenate(probs_c, axis=0),
            jnp.concatenate(w_c, axis=0),
            jnp.concatenate(i_c, axis=0))


# speedup vs baseline: 1.0778x; 1.0114x over previous
import functools
import numpy as np

import jax
import jax.numpy as jnp
from jax import lax
from jax.experimental import pallas as pl
from jax.experimental.pallas import tpu as pltpu
from jax.experimental.pallas import tpu_sc as plsc

D_MODEL = 4096
N_EXPERTS = 64
TOP_K = 8
N_CAND = 32
EPS = 1e-6
TB = 1024
N_TOKENS = 8192
NW = 32
NCHUNKS = 1
CT = N_TOKENS // NCHUNKS
NEG = -3.4e38


def _tc_body(h_ref, weff_ref, wcat_ref, b_ref, probs_ref, bvh_ref):
    x = h_ref[...]
    var = jnp.mean(x * x, axis=-1, keepdims=True)
    xn = x * lax.rsqrt(var + EPS) * weff_ref[...]
    logits = jnp.dot(xn, wcat_ref[...], preferred_element_type=jnp.float32)
    bvh_ref[...] = logits[:, :N_EXPERTS]
    rl = logits[:, N_EXPERTS:] + b_ref[...]
    m = jnp.max(rl, axis=-1, keepdims=True)
    e = jnp.exp(rl - m)
    probs_ref[...] = e / jnp.sum(e, axis=-1, keepdims=True)


def _rev(x):
    return lax.rev(x, dimensions=(0,))


def _vsort(x):
    return lax.sort(x, dimension=0, is_stable=False, num_keys=1)


def _make_sc_body(tpw):
    def _sc_body(bvh_hbm, probs_hbm, pes_hbm, w_hbm, i_hbm,
                 bvh_v, probs_v, pes_v, w_v, i_v):
        wid = lax.axis_index("s") * 2 + lax.axis_index("c")
        base = wid * tpw * N_EXPERTS
        obase = wid * tpw * 16
        pltpu.sync_copy(bvh_hbm.at[pl.ds(base, tpw * N_EXPERTS)], bvh_v)
        pltpu.sync_copy(probs_hbm.at[pl.ds(base, tpw * N_EXPERTS)], probs_v)
        pltpu.sync_copy(pes_hbm, pes_v)

        iota = lax.broadcasted_iota(jnp.int32, (16,), 0)
        ids = [iota + (16 * k) for k in range(4)]

        @plsc.parallel_loop(0, tpw, 1, unroll=2)
        def token_body(t):
            tb = t * N_EXPERTS
            b = [bvh_v[pl.ds(tb + 16 * k, 16)] for k in range(4)]
            g = [probs_v[pl.ds(tb + 16 * k, 16)] for k in range(4)]

            s = [_vsort(b[k]) for k in range(4)]
            lo01 = _vsort(jnp.minimum(s[0], _rev(s[1])))
            hi01 = _vsort(jnp.maximum(s[0], _rev(s[1])))
            lo23 = _vsort(jnp.minimum(s[2], _rev(s[3])))
            hi23 = _vsort(jnp.maximum(s[2], _rev(s[3])))
            u0 = jnp.maximum(lo01, _rev(hi23))
            u1 = jnp.maximum(hi01, _rev(lo23))
            thr = jnp.min(jnp.minimum(u0, u1))

            p = [jnp.where(b[k] >= thr, g[k], -1.0) for k in range(4)]
            tk = []
            for k in range(4):
                si = plsc.bitcast(b[k], jnp.int32)
                key = si ^ jnp.bitwise_and(si >> 31, jnp.int32(0x7FFFFFFF))
                tk.append(jnp.bitwise_or(jnp.bitwise_and(key, jnp.int32(-64)),
                                         ids[k]))
            imin = jnp.int32(-2147483648)
            out_v = jnp.zeros((16,), jnp.float32)
            out_i = jnp.zeros((16,), jnp.int32)
            for r in range(TOP_K):
                mx = jnp.max(jnp.maximum(jnp.maximum(p[0], p[1]),
                                         jnp.maximum(p[2], p[3])))
                tb_ = [jnp.where(p[k] == mx, tk[k], imin) for k in range(4)]
                tm = jnp.max(jnp.maximum(jnp.maximum(tb_[0], tb_[1]),
                                         jnp.maximum(tb_[2], tb_[3])))
                sel2 = [tk[k] == tm for k in range(4)]
                slot = iota == r
                out_v = jnp.where(slot, mx, out_v)
                out_i = jnp.where(slot, jnp.bitwise_and(tm, 63), out_i)
                p = [jnp.where(sel2[k], -1.0, p[k]) for k in range(4)]

            valid = iota < TOP_K
            total = jnp.sum(jnp.where(valid, out_v, 0.0))
            pes_g = plsc.load_gather(pes_v, [out_i])
            w = out_v / total * pes_g
            w_v[pl.ds(t * 16, 16)] = w
            i_v[pl.ds(t * 16, 16)] = out_i

        pltpu.sync_copy(w_v, w_hbm.at[pl.ds(obase, tpw * 16)])
        pltpu.sync_copy(i_v, i_hbm.at[pl.ds(obase, tpw * 16)])

    return _sc_body


def _make_sc_route(nt):
    tpw = nt // NW
    return pl.kernel(
        _make_sc_body(tpw),
        mesh=plsc.VectorSubcoreMesh(core_axis_name="c", subcore_axis_name="s"),
        out_type=[
            jax.ShapeDtypeStruct((nt * 16,), jnp.float32),
            jax.ShapeDtypeStruct((nt * 16,), jnp.int32),
        ],
        scratch_types=[
            pltpu.VMEM((tpw * N_EXPERTS,), jnp.float32),
            pltpu.VMEM((tpw * N_EXPERTS,), jnp.float32),
            pltpu.VMEM((N_EXPERTS,), jnp.float32),
            pltpu.VMEM((tpw * 16,), jnp.float32),
            pltpu.VMEM((tpw * 16,), jnp.int32),
        ],
        compiler_params=pltpu.CompilerParams(needs_layout_passes=False),
    )


_sc_route = _make_sc_route(CT)


def _tc_dense(h, weff, wcat, b2):
    nt = h.shape[0]
    grid = (nt // TB,)
    return pl.pallas_call(
        _tc_body,
        grid=grid,
        in_specs=[
            pl.BlockSpec((TB, D_MODEL), lambda i: (i, 0)),
            pl.BlockSpec((1, D_MODEL), lambda i: (0, 0)),
            pl.BlockSpec((D_MODEL, 2 * N_EXPERTS), lambda i: (0, 0)),
            pl.BlockSpec((1, N_EXPERTS), lambda i: (0, 0)),
        ],
        out_specs=[
            pl.BlockSpec((TB, N_EXPERTS), lambda i: (i, 0)),
            pl.BlockSpec((TB, N_EXPERTS), lambda i: (i, 0)),
        ],
        out_shape=[
            jax.ShapeDtypeStruct((nt, N_EXPERTS), jnp.float32),
            jax.ShapeDtypeStruct((nt, N_EXPERTS), jnp.float32),
        ],
    )(h, weff, wcat, b2)


@jax.jit
def kernel(hidden_states, norm_weight, scale, W_proj, b_proj, W_bvh,
           per_expert_scale):
    weff = (norm_weight * scale * np.float32(np.sqrt(D_MODEL))).reshape(1, D_MODEL)
    wcat = jnp.concatenate([W_bvh, W_proj], axis=1)
    b2 = b_proj.reshape(1, N_EXPERTS)

    probs_c, w_c, i_c = [], [], []
    dense = []
    for c in range(NCHUNKS):
        h = lax.slice_in_dim(hidden_states, c * CT, (c + 1) * CT, axis=0)
        dense.append(_tc_dense(h, weff, wcat, b2))
    for c in range(NCHUNKS):
        probs, bvh = dense[c]
        w16, i16 = _sc_route(bvh.reshape(-1), probs.reshape(-1),
                             per_expert_scale)
        probs_c.append(probs)
        w_c.append(w16.reshape(CT, 16)[:, :TOP_K])
        i_c.append(i16.reshape(CT, 16)[:, :TOP_K])

    return (jnp.concatenate(probs_c, axis=0),
            jnp.concatenate(w_c, axis=0),
            jnp.concatenate(i_c, axis=0))
